# baseline (speedup 1.0000x reference)
import functools

import jax
import jax.numpy as jnp
from jax import lax
from jax.experimental import pallas as pl
from jax.experimental.pallas import tpu as pltpu
from jax.experimental.pallas import tpu_sc as plsc

B, F, D = 128, 16, 1000
L = 16
NC, NS = 2, 16
NW = NC * NS
BPW = B // NW
NCHUNK = D // L

_mesh = plsc.VectorSubcoreMesh(core_axis_name="c", subcore_axis_name="s")


def _sum_chunk(x_v, o_v, sl):
    acc = x_v[0, sl] + x_v[1, sl]
    for f in range(2, F):
        acc = acc + x_v[f, sl]
    o_v[sl] = acc * (1.0 / F)


@functools.partial(
    pl.kernel,
    mesh=_mesh,
    out_type=jax.ShapeDtypeStruct((B, D), jnp.float32),
    scratch_types=[
        pltpu.VMEM((F, D), jnp.float32),
        pltpu.VMEM((D,), jnp.float32),
    ],
)
def _mean_sc(x_hbm, out_hbm, x_v, o_v):
    wid = lax.axis_index("s") * NC + lax.axis_index("c")
    for j in range(BPW):
        b = wid * BPW + j
        pltpu.sync_copy(x_hbm.at[b], x_v)

        def chunk(i, carry):
            _sum_chunk(x_v, o_v, pl.ds(pl.multiple_of(i * L, L), L))
            return carry

        lax.fori_loop(0, NCHUNK, chunk, 0)
        _sum_chunk(x_v, o_v, pl.ds(D - L, L))
        pltpu.sync_copy(o_v, out_hbm.at[b])


def kernel(input):
    return _mean_sc(input).reshape(B, 1, D)


# speedup vs baseline: 1.0863x; 1.0863x over previous
import functools

import jax
import jax.numpy as jnp
from jax import lax
from jax.experimental import pallas as pl
from jax.experimental.pallas import tpu as pltpu
from jax.experimental.pallas import tpu_sc as plsc

B, F, D = 128, 16, 1000
L = 16
NC, NS = 2, 16
NW = NC * NS
BPW = B // NW
UNROLL = 2
NITER = (D // L) // UNROLL

_mesh = plsc.VectorSubcoreMesh(core_axis_name="c", subcore_axis_name="s")


def _sum_chunk(x_v, o_v, j, sl):
    acc0 = x_v[j, 0, sl] + x_v[j, 1, sl]
    acc1 = x_v[j, 2, sl] + x_v[j, 3, sl]
    acc2 = x_v[j, 4, sl] + x_v[j, 5, sl]
    acc3 = x_v[j, 6, sl] + x_v[j, 7, sl]
    for f in range(8, F):
        acc0 = acc0 + x_v[j, f, sl]
    o_v[j, sl] = ((acc0 + acc1) + (acc2 + acc3)) * (1.0 / F)


@functools.partial(
    pl.kernel,
    mesh=_mesh,
    out_type=jax.ShapeDtypeStruct((B, D), jnp.float32),
    scratch_types=[
        pltpu.VMEM((BPW, F, D), jnp.float32),
        pltpu.VMEM((BPW, D), jnp.float32),
        pltpu.SemaphoreType.DMA,
        pltpu.SemaphoreType.DMA,
        pltpu.SemaphoreType.DMA,
        pltpu.SemaphoreType.DMA,
        pltpu.SemaphoreType.DMA,
    ],
)
def _mean_sc(x_hbm, out_hbm, x_v, o_v, s0, s1, s2, s3, so):
    wid = lax.axis_index("s") * NC + lax.axis_index("c")
    base = wid * BPW
    sems = (s0, s1, s2, s3)
    copies = [
        pltpu.async_copy(x_hbm.at[base + j], x_v.at[j], sems[j])
        for j in range(BPW)
    ]
    out_copies = []
    for j in range(BPW):
        copies[j].wait()

        def chunk(i, carry):
            for u in range(UNROLL):
                off = pl.multiple_of(i * (UNROLL * L) + u * L, L)
                _sum_chunk(x_v, o_v, j, pl.ds(off, L))
            return carry

        lax.fori_loop(0, NITER, chunk, 0)
        _sum_chunk(x_v, o_v, j, pl.ds(D - L, L))
        out_copies.append(pltpu.async_copy(o_v.at[j], out_hbm.at[base + j], so))
    for cp in out_copies:
        cp.wait()


def kernel(input):
    return _mean_sc(input).reshape(B, 1, D)


# speedup vs baseline: 1.0895x; 1.0030x over previous
import functools

import jax
import jax.numpy as jnp
from jax import lax
from jax.experimental import pallas as pl
from jax.experimental.pallas import tpu as pltpu
from jax.experimental.pallas import tpu_sc as plsc

B, F, D = 128, 16, 1000
L = 16
NC, NS = 2, 16
NW = NC * NS
BPW = B // NW
UNROLL = 2
NITER = (D // L) // UNROLL

_mesh = plsc.VectorSubcoreMesh(core_axis_name="c", subcore_axis_name="s")


def _sum_chunk(x_v, o_v, j, sl):
    acc0 = x_v[j, 0, sl] + x_v[j, 1, sl]
    acc1 = x_v[j, 2, sl] + x_v[j, 3, sl]
    acc2 = x_v[j, 4, sl] + x_v[j, 5, sl]
    acc3 = x_v[j, 6, sl] + x_v[j, 7, sl]
    for f in range(8, F):
        acc0 = acc0 + x_v[j, f, sl]
    o_v[j, sl] = ((acc0 + acc1) + (acc2 + acc3)) * (1.0 / F)


@functools.partial(
    pl.kernel,
    mesh=_mesh,
    out_type=jax.ShapeDtypeStruct((B, D), jnp.float32),
    compiler_params=pltpu.CompilerParams(use_tc_tiling_on_sc=True),
    scratch_types=[
        pltpu.VMEM((BPW, F, D), jnp.float32),
        pltpu.VMEM((BPW, D), jnp.float32),
        pltpu.SemaphoreType.DMA,
        pltpu.SemaphoreType.DMA,
        pltpu.SemaphoreType.DMA,
        pltpu.SemaphoreType.DMA,
        pltpu.SemaphoreType.DMA,
    ],
)
def _mean_sc(x_hbm, out_hbm, x_v, o_v, s0, s1, s2, s3, so):
    wid = lax.axis_index("s") * NC + lax.axis_index("c")
    base = wid * BPW
    sems = (s0, s1, s2, s3)
    copies = [
        pltpu.async_copy(x_hbm.at[base + j], x_v.at[j], sems[j])
        for j in range(BPW)
    ]
    out_copies = []
    for j in range(BPW):
        copies[j].wait()

        def chunk(i, carry):
            for u in range(UNROLL):
                off = pl.multiple_of(i * (UNROLL * L) + u * L, L)
                _sum_chunk(x_v, o_v, j, pl.ds(off, L))
            return carry

        lax.fori_loop(0, NITER, chunk, 0)
        _sum_chunk(x_v, o_v, j, pl.ds(D - L, L))
        out_copies.append(pltpu.async_copy(o_v.at[j], out_hbm.at[base + j], so))
    for cp in out_copies:
        cp.wait()


def kernel(input):
    return _mean_sc(input).reshape(B, 1, D)


# speedup vs baseline: 1.5372x; 1.4109x over previous
import functools

import jax
import jax.numpy as jnp
from jax import lax
from jax.experimental import pallas as pl
from jax.experimental.pallas import tpu as pltpu
from jax.experimental.pallas import tpu_sc as plsc

B, F, D = 128, 16, 1000
L = 16
NC, NS = 2, 16
NW = NC * NS
TP = 32

_mesh = plsc.VectorSubcoreMesh(core_axis_name="c", subcore_axis_name="s")


@functools.partial(
    pl.kernel,
    mesh=_mesh,
    out_type=jax.ShapeDtypeStruct((D, B), jnp.float32),
    scratch_types=[
        pltpu.VMEM((F, TP, B), jnp.float32),
        pltpu.VMEM((TP, B), jnp.float32),
        pltpu.SemaphoreType.DMA,
    ],
)
def _mean_sc(x_hbm, out_hbm, x_v, o_v, sem):
    wid = lax.axis_index("s") * NC + lax.axis_index("c")
    tile = jnp.minimum(wid * (TP // 8), D // 8 - TP // 8)
    start = pl.multiple_of(tile * 8, 8)
    copies = [
        pltpu.async_copy(x_hbm.at[f, pl.ds(start, TP), :], x_v.at[f], sem)
        for f in range(F)
    ]
    for cp in copies:
        cp.wait()

    def row(r, carry):
        for c in range(B // L):
            sl = pl.ds(c * L, L)
            acc0 = x_v[0, r, sl] + x_v[1, r, sl]
            acc1 = x_v[2, r, sl] + x_v[3, r, sl]
            acc2 = x_v[4, r, sl] + x_v[5, r, sl]
            acc3 = x_v[6, r, sl] + x_v[7, r, sl]
            for f in range(8, F):
                acc0 = acc0 + x_v[f, r, sl]
            o_v[r, sl] = ((acc0 + acc1) + (acc2 + acc3)) * (1.0 / F)
        return carry

    lax.fori_loop(0, TP, row, 0)
    pltpu.sync_copy(o_v, out_hbm.at[pl.ds(start, TP), :])


def kernel(input):
    x_t = jnp.transpose(input, (1, 2, 0))
    out_t = _mean_sc(x_t)
    return jnp.transpose(out_t)[:, None, :]
